# Initial kernel scaffold; baseline (speedup 1.0000x reference)
import jax
import jax.numpy as jnp
from jax.experimental import pallas as pl


def kernel(x, edge_index, W_l1, W_r1, b1, W_l2, W_r2, b2):
    raise NotImplementedError("write your pallas kernel here")


# speedup vs baseline: 3.6404x; 3.6404x over previous
import functools

import jax
import jax.numpy as jnp
from jax import lax
from jax.experimental import pallas as pl
from jax.experimental.pallas import tpu as pltpu
from jax.experimental.pallas import tpu_sc as plsc

N, E, DIN, H, DOUT = 10000, 320000, 128, 256, 300
N_PAD = 12800
E_PAD = 323584
CHUNK = 128
NSUB = 16
NCORE = 2
ROWS_PER_SUB = N_PAD // NSUB
CNT_W = 16
BLK = 200
GRID_R = N // BLK
COFF = N_PAD // BLK

_MESH = plsc.VectorSubcoreMesh(core_axis_name="c", subcore_axis_name="s")


def _zero_acc_rows(rows_v, acc, base_r):
    full, rem = divmod(ROWS_PER_SUB, CHUNK)
    for t in range(full):
        pltpu.sync_copy(rows_v, acc.at[pl.ds(base_r + t * CHUNK, CHUNK)])
    if rem:
        pltpu.sync_copy(rows_v.at[pl.ds(0, rem)],
                        acc.at[pl.ds(base_r + full * CHUNK, rem)])


@functools.partial(
    pl.kernel,
    out_type=[jax.ShapeDtypeStruct((NCORE * N_PAD, DIN), jnp.float32),
              jax.ShapeDtypeStruct((NCORE * N_PAD,), jnp.float32)],
    mesh=_MESH,
    scratch_types=[
        pltpu.VMEM((CHUNK,), jnp.int32),
        pltpu.VMEM((1, CHUNK), jnp.int32),
        pltpu.VMEM((CHUNK, DIN), jnp.float32),
        pltpu.VMEM((CHUNK,), jnp.float32),
        pltpu.VMEM((ROWS_PER_SUB,), jnp.float32),
        pltpu.VMEM_SHARED((N_PAD, DIN), jnp.float32),
        pltpu.VMEM_SHARED((N_PAD,), jnp.float32),
        pltpu.SemaphoreType.DMA,
    ],
)
def _sc_seg1(x_hbm, src_hbm, dst_hbm, psum_hbm, pcnt_hbm,
             src_v, dst_v, rows_v, ones_v, zc_v, acc, cnt_sh, sem):
    c = lax.axis_index("c")
    s = lax.axis_index("s")

    for k in range(CHUNK // 16):
        ones_v[pl.ds(k * 16, 16)] = jnp.ones((16,), jnp.float32)

    def _fill(i, carry):
        for k in range(DIN // 16):
            rows_v[i, pl.ds(k * 16, 16)] = jnp.zeros((16,), jnp.float32)
        return carry
    lax.fori_loop(0, CHUNK, _fill, None)

    def _fill_zc(i, carry):
        zc_v[pl.ds(i * 16, 16)] = jnp.zeros((16,), jnp.float32)
        return carry
    lax.fori_loop(0, ROWS_PER_SUB // 16, _fill_zc, None)

    base_r = s * ROWS_PER_SUB
    _zero_acc_rows(rows_v, acc, base_r)
    pltpu.sync_copy(zc_v, cnt_sh.at[pl.ds(base_r, ROWS_PER_SUB)])
    plsc.subcore_barrier()

    ew = E_PAD // (NCORE * NSUB)
    w = c * NSUB + s

    def _body(j, carry):
        base = w * ew + j * CHUNK
        pltpu.sync_copy(src_hbm.at[pl.ds(base, CHUNK)], src_v)
        pltpu.sync_copy(dst_hbm.at[pl.ds(base, CHUNK)], dst_v.at[0])
        pltpu.async_copy(x_hbm.at[src_v], rows_v, sem).wait()
        pltpu.sync_copy(rows_v, acc.at[dst_v.at[0]], add=True)
        pltpu.sync_copy(ones_v, cnt_sh.at[dst_v.at[0]], add=True)
        return carry
    lax.fori_loop(0, ew // CHUNK, _body, None)

    plsc.subcore_barrier()
    out_r0 = c * N_PAD + base_r
    pltpu.sync_copy(acc.at[pl.ds(base_r, ROWS_PER_SUB)],
                    psum_hbm.at[pl.ds(out_r0, ROWS_PER_SUB)])
    pltpu.sync_copy(cnt_sh.at[pl.ds(base_r, ROWS_PER_SUB)], zc_v)
    pltpu.sync_copy(zc_v, pcnt_hbm.at[pl.ds(out_r0, ROWS_PER_SUB)])


@functools.partial(
    pl.kernel,
    out_type=jax.ShapeDtypeStruct((N_PAD, H), jnp.float32),
    mesh=_MESH,
    scratch_types=[
        pltpu.VMEM((CHUNK,), jnp.int32),
        pltpu.VMEM((1, CHUNK), jnp.int32),
        pltpu.VMEM((CHUNK, 128), jnp.float32),
        pltpu.VMEM_SHARED((N_PAD, 128), jnp.float32),
        pltpu.SemaphoreType.DMA,
    ],
)
def _sc_seg2(h_hbm, src2_hbm, dst_hbm, summ2_hbm,
             src_v, dst_v, rows_v, acc, sem):
    c = lax.axis_index("c")
    s = lax.axis_index("s")

    def _fill(i, carry):
        for k in range(128 // 16):
            rows_v[i, pl.ds(k * 16, 16)] = jnp.zeros((16,), jnp.float32)
        return carry
    lax.fori_loop(0, CHUNK, _fill, None)

    base_r = s * ROWS_PER_SUB
    _zero_acc_rows(rows_v, acc, base_r)
    plsc.subcore_barrier()

    es = E_PAD // NSUB

    def _body(j, carry):
        base = s * es + j * CHUNK
        pltpu.sync_copy(src2_hbm.at[pl.ds(c * E_PAD + base, CHUNK)], src_v)
        pltpu.sync_copy(dst_hbm.at[pl.ds(base, CHUNK)], dst_v.at[0])
        pltpu.async_copy(h_hbm.at[src_v], rows_v, sem).wait()
        pltpu.sync_copy(rows_v, acc.at[dst_v.at[0]], add=True)
        return carry
    lax.fori_loop(0, es // CHUNK, _body, None)

    plsc.subcore_barrier()
    pltpu.sync_copy(acc.at[pl.ds(base_r, ROWS_PER_SUB)],
                    summ2_hbm.at[pl.ds(base_r, ROWS_PER_SUB),
                                 pl.ds(c * 128, 128)])


def _tc1_body(ps0, ps1, pc0, pc1, x_r, wl_r, wr_r, b_r, h_r):
    cnt = jnp.maximum(pc0[...] + pc1[...], 1.0)
    mean = (ps0[...] + ps1[...]) / cnt
    h = jnp.dot(mean, wl_r[...], preferred_element_type=jnp.float32)
    h += jnp.dot(x_r[...], wr_r[...], preferred_element_type=jnp.float32)
    h_r[...] = jnp.maximum(h + b_r[...], 0.0)


def _tc_layer1(psum, pcnt, x, W_l1, W_r1, b1r):
    return pl.pallas_call(
        _tc1_body,
        grid=(GRID_R, 2),
        in_specs=[
            pl.BlockSpec((BLK, DIN), lambda i, j: (i, 0)),
            pl.BlockSpec((BLK, DIN), lambda i, j: (COFF + i, 0)),
            pl.BlockSpec((BLK, 1), lambda i, j: (i, 0)),
            pl.BlockSpec((BLK, 1), lambda i, j: (COFF + i, 0)),
            pl.BlockSpec((BLK, DIN), lambda i, j: (i, 0)),
            pl.BlockSpec((DIN, 128), lambda i, j: (0, j)),
            pl.BlockSpec((DIN, 128), lambda i, j: (0, j)),
            pl.BlockSpec((1, 128), lambda i, j: (0, j)),
        ],
        out_specs=pl.BlockSpec((BLK, 128), lambda i, j: (j * COFF + i, 0)),
        out_shape=jax.ShapeDtypeStruct((NCORE * N_PAD, 128), jnp.float32),
    )(psum, psum, pcnt, pcnt, x, W_l1, W_r1, b1r)


def _tc2_body(s2, pc0, pc1, hl, hr, wl, wra, wrb, b_r, o_r):
    cnt = jnp.maximum(pc0[...] + pc1[...], 1.0)
    mean = s2[...] / cnt
    z = jnp.dot(mean, wl[...], preferred_element_type=jnp.float32)
    z += jnp.dot(hl[...], wra[...], preferred_element_type=jnp.float32)
    z += jnp.dot(hr[...], wrb[...], preferred_element_type=jnp.float32)
    z += b_r[...]
    m = jnp.max(z, axis=1, keepdims=True)
    ez = jnp.exp(z - m)
    o_r[...] = (z - m) - jnp.log(jnp.sum(ez, axis=1, keepdims=True))


def _tc_layer2(summ2, pcnt, h2, W_l2, W_r2a, W_r2b, b2r):
    return pl.pallas_call(
        _tc2_body,
        grid=(GRID_R,),
        in_specs=[
            pl.BlockSpec((BLK, H), lambda i: (i, 0)),
            pl.BlockSpec((BLK, 1), lambda i: (i, 0)),
            pl.BlockSpec((BLK, 1), lambda i: (COFF + i, 0)),
            pl.BlockSpec((BLK, 128), lambda i: (i, 0)),
            pl.BlockSpec((BLK, 128), lambda i: (COFF + i, 0)),
            pl.BlockSpec((H, DOUT), lambda i: (0, 0)),
            pl.BlockSpec((128, DOUT), lambda i: (0, 0)),
            pl.BlockSpec((128, DOUT), lambda i: (0, 0)),
            pl.BlockSpec((1, DOUT), lambda i: (0, 0)),
        ],
        out_specs=pl.BlockSpec((BLK, DOUT), lambda i: (i, 0)),
        out_shape=jax.ShapeDtypeStruct((N, DOUT), jnp.float32),
    )(summ2, pcnt, pcnt, h2, h2, W_l2, W_r2a, W_r2b, b2r)


def kernel(x, edge_index, W_l1, W_r1, b1, W_l2, W_r2, b2):
    src = edge_index[0]
    dst = edge_index[1]
    pad = jnp.full((E_PAD - E,), N, jnp.int32)
    src_p = jnp.concatenate([src, pad])
    dst_p = jnp.concatenate([dst, pad])
    src2_p = jnp.concatenate([src_p, src_p + N_PAD])
    x_pad = jnp.zeros((N_PAD, DIN), jnp.float32).at[:N].set(x)

    psum, pcnt = _sc_seg1(x_pad, src_p, dst_p)
    pcnt = pcnt.reshape(NCORE * N_PAD, 1)
    h2 = _tc_layer1(psum, pcnt, x, W_l1, W_r1, b1.reshape(1, H))
    summ2 = _sc_seg2(h2, src2_p, dst_p)
    return _tc_layer2(summ2, pcnt, h2, W_l2, W_r2[:128], W_r2[128:],
                      b2.reshape(1, DOUT))
